# baseline (speedup 1.0000x reference)
import functools

import jax
import jax.numpy as jnp
from jax import lax
from jax.experimental import pallas as pl
from jax.experimental.pallas import tpu as pltpu
from jax.experimental.pallas import tpu_sc as plsc

N_Q = 32
N_KV = 8192
D_HEAD = 128
MAX_OFFSET = 512

S_ROWS = 2688
ROWS_PER_TILE = S_ROWS // 16
HEAD = 576
CONST_START = 608
CONST_LEN = 2048
TAIL = N_KV - HEAD
N_FULL = TAIL // CONST_LEN
REM = TAIL % CONST_LEN
FIRE = 13


def _make_rpe():
    mesh = plsc.VectorSubcoreMesh(core_axis_name="c", subcore_axis_name="s")

    @functools.partial(
        pl.kernel,
        mesh=mesh,
        out_type=jax.ShapeDtypeStruct((N_Q, N_KV, D_HEAD), jnp.float32),
        scratch_types=[
            pltpu.VMEM_SHARED((S_ROWS, D_HEAD), jnp.float32),
            pltpu.SemaphoreType.DMA,
            pltpu.SemaphoreType.DMA,
        ],
    )
    def rpe(table_hbm, out_hbm, s_ref, bsem, wsem):
        c = lax.axis_index("c")
        s = lax.axis_index("s")
        q = c * 16 + s

        u0 = s * ROWS_PER_TILE
        for base in range(0, ROWS_PER_TILE, FIRE):
            burst = [
                pltpu.async_copy(
                    table_hbm.at[jnp.maximum(543 - (u0 + base + r), 0)],
                    s_ref.at[u0 + base + r],
                    bsem,
                )
                for r in range(min(FIRE, ROWS_PER_TILE - base))
            ]
            for cp in burst:
                cp.wait()

        plsc.subcore_barrier()

        writes = [
            pltpu.async_copy(
                s_ref.at[pl.ds(31 - q, HEAD)],
                out_hbm.at[q, pl.ds(0, HEAD)],
                wsem,
            )
        ]
        const_src = s_ref.at[pl.ds(CONST_START, CONST_LEN)]
        for i in range(N_FULL):
            writes.append(
                pltpu.async_copy(
                    const_src,
                    out_hbm.at[q, pl.ds(HEAD + i * CONST_LEN, CONST_LEN)],
                    wsem,
                )
            )
        if REM:
            writes.append(
                pltpu.async_copy(
                    s_ref.at[pl.ds(CONST_START, REM)],
                    out_hbm.at[q, pl.ds(N_KV - REM, REM)],
                    wsem,
                )
            )
        for cp in writes:
            cp.wait()

    return rpe


_rpe = _make_rpe()


def kernel(n_q, n_kv, pos_emb):
    del n_q, n_kv
    return _rpe(pos_emb)


# speedup vs baseline: 1.9926x; 1.9926x over previous
import functools

import jax
import jax.numpy as jnp
from jax import lax
from jax.experimental import pallas as pl
from jax.experimental.pallas import tpu as pltpu
from jax.experimental.pallas import tpu_sc as plsc

N_Q = 32
N_KV = 8192
D_HEAD = 128
MAX_OFFSET = 512

S_ROWS = 1120
ROWS_PER_TILE = S_ROWS // 16
HEAD = 576
CONST_START = 608
CONST_LEN = 512
TAIL = N_KV - HEAD
N_FULL = TAIL // CONST_LEN
REM = TAIL % CONST_LEN
FIRE = 13


def _make_rpe():
    mesh = plsc.VectorSubcoreMesh(core_axis_name="c", subcore_axis_name="s")

    @functools.partial(
        pl.kernel,
        mesh=mesh,
        out_type=jax.ShapeDtypeStruct((N_Q, N_KV, D_HEAD), jnp.float32),
        scratch_types=[
            pltpu.VMEM_SHARED((S_ROWS, D_HEAD), jnp.float32),
            pltpu.SemaphoreType.DMA,
            pltpu.SemaphoreType.DMA,
        ],
    )
    def rpe(table_hbm, out_hbm, s_ref, bsem, wsem):
        c = lax.axis_index("c")
        s = lax.axis_index("s")
        q = c * 16 + s

        u0 = s * ROWS_PER_TILE
        for base in range(0, ROWS_PER_TILE, FIRE):
            burst = [
                pltpu.async_copy(
                    table_hbm.at[jnp.maximum(543 - (u0 + base + r), 0)],
                    s_ref.at[u0 + base + r],
                    bsem,
                )
                for r in range(min(FIRE, ROWS_PER_TILE - base))
            ]
            for cp in burst:
                cp.wait()

        plsc.subcore_barrier()

        writes = [
            pltpu.async_copy(
                s_ref.at[pl.ds(31 - q, HEAD)],
                out_hbm.at[q, pl.ds(0, HEAD)],
                wsem,
            )
        ]
        const_src = s_ref.at[pl.ds(CONST_START, CONST_LEN)]
        for i in range(N_FULL):
            writes.append(
                pltpu.async_copy(
                    const_src,
                    out_hbm.at[q, pl.ds(HEAD + i * CONST_LEN, CONST_LEN)],
                    wsem,
                )
            )
        if REM:
            writes.append(
                pltpu.async_copy(
                    s_ref.at[pl.ds(CONST_START, REM)],
                    out_hbm.at[q, pl.ds(N_KV - REM, REM)],
                    wsem,
                )
            )
        for cp in writes:
            cp.wait()

    return rpe


_rpe = _make_rpe()


def kernel(n_q, n_kv, pos_emb):
    del n_q, n_kv
    return _rpe(pos_emb)


# speedup vs baseline: 2.3515x; 1.1801x over previous
import functools

import jax
import jax.numpy as jnp
from jax import lax
from jax.experimental import pallas as pl
from jax.experimental.pallas import tpu as pltpu
from jax.experimental.pallas import tpu_sc as plsc

N_Q = 32
N_KV = 8192
D_HEAD = 128
MAX_OFFSET = 512

S_ROWS = 864
ROWS_PER_TILE = S_ROWS // 16
HEAD = 576
CONST_START = 608
CONST_LEN = 256
TAIL = N_KV - HEAD
N_FULL = TAIL // CONST_LEN
REM = TAIL % CONST_LEN
FIRE = 13


def _make_rpe():
    mesh = plsc.VectorSubcoreMesh(core_axis_name="c", subcore_axis_name="s")

    @functools.partial(
        pl.kernel,
        mesh=mesh,
        out_type=jax.ShapeDtypeStruct((N_Q, N_KV, D_HEAD), jnp.float32),
        scratch_types=[
            pltpu.VMEM_SHARED((S_ROWS, D_HEAD), jnp.float32),
            pltpu.SemaphoreType.DMA,
            pltpu.SemaphoreType.DMA,
        ],
    )
    def rpe(table_hbm, out_hbm, s_ref, bsem, wsem):
        c = lax.axis_index("c")
        s = lax.axis_index("s")
        q = c * 16 + s

        u0 = s * ROWS_PER_TILE
        for base in range(0, ROWS_PER_TILE, FIRE):
            burst = [
                pltpu.async_copy(
                    table_hbm.at[jnp.maximum(543 - (u0 + base + r), 0)],
                    s_ref.at[u0 + base + r],
                    bsem,
                )
                for r in range(min(FIRE, ROWS_PER_TILE - base))
            ]
            for cp in burst:
                cp.wait()

        plsc.subcore_barrier()

        writes = [
            pltpu.async_copy(
                s_ref.at[pl.ds(31 - q, HEAD)],
                out_hbm.at[q, pl.ds(0, HEAD)],
                wsem,
            )
        ]
        const_src = s_ref.at[pl.ds(CONST_START, CONST_LEN)]
        for i in range(N_FULL):
            writes.append(
                pltpu.async_copy(
                    const_src,
                    out_hbm.at[q, pl.ds(HEAD + i * CONST_LEN, CONST_LEN)],
                    wsem,
                )
            )
        if REM:
            writes.append(
                pltpu.async_copy(
                    s_ref.at[pl.ds(CONST_START, REM)],
                    out_hbm.at[q, pl.ds(N_KV - REM, REM)],
                    wsem,
                )
            )
        for cp in writes:
            cp.wait()

    return rpe


_rpe = _make_rpe()


def kernel(n_q, n_kv, pos_emb):
    del n_q, n_kv
    return _rpe(pos_emb)


# speedup vs baseline: 2.4868x; 1.0576x over previous
import functools

import jax
import jax.numpy as jnp
from jax import lax
from jax.experimental import pallas as pl
from jax.experimental.pallas import tpu as pltpu
from jax.experimental.pallas import tpu_sc as plsc

N_Q = 32
N_KV = 8192
D_HEAD = 128
MAX_OFFSET = 512

S_ROWS = 736
ROWS_PER_TILE = S_ROWS // 16
HEAD = 576
CONST_START = 608
CONST_LEN = 128
TAIL = N_KV - HEAD
N_FULL = TAIL // CONST_LEN
REM = TAIL % CONST_LEN
FIRE = 13


def _make_rpe():
    mesh = plsc.VectorSubcoreMesh(core_axis_name="c", subcore_axis_name="s")

    @functools.partial(
        pl.kernel,
        mesh=mesh,
        out_type=jax.ShapeDtypeStruct((N_Q, N_KV, D_HEAD), jnp.float32),
        scratch_types=[
            pltpu.VMEM_SHARED((S_ROWS, D_HEAD), jnp.float32),
            pltpu.SemaphoreType.DMA,
            pltpu.SemaphoreType.DMA,
        ],
    )
    def rpe(table_hbm, out_hbm, s_ref, bsem, wsem):
        c = lax.axis_index("c")
        s = lax.axis_index("s")
        q = c * 16 + s

        u0 = s * ROWS_PER_TILE
        for base in range(0, ROWS_PER_TILE, FIRE):
            burst = [
                pltpu.async_copy(
                    table_hbm.at[jnp.maximum(543 - (u0 + base + r), 0)],
                    s_ref.at[u0 + base + r],
                    bsem,
                )
                for r in range(min(FIRE, ROWS_PER_TILE - base))
            ]
            for cp in burst:
                cp.wait()

        plsc.subcore_barrier()

        writes = [
            pltpu.async_copy(
                s_ref.at[pl.ds(31 - q, HEAD)],
                out_hbm.at[q, pl.ds(0, HEAD)],
                wsem,
            )
        ]
        const_src = s_ref.at[pl.ds(CONST_START, CONST_LEN)]
        for i in range(N_FULL):
            writes.append(
                pltpu.async_copy(
                    const_src,
                    out_hbm.at[q, pl.ds(HEAD + i * CONST_LEN, CONST_LEN)],
                    wsem,
                )
            )
        if REM:
            writes.append(
                pltpu.async_copy(
                    s_ref.at[pl.ds(CONST_START, REM)],
                    out_hbm.at[q, pl.ds(N_KV - REM, REM)],
                    wsem,
                )
            )
        for cp in writes:
            cp.wait()

    return rpe


_rpe = _make_rpe()


def kernel(n_q, n_kv, pos_emb):
    del n_q, n_kv
    return _rpe(pos_emb)


# speedup vs baseline: 2.6533x; 1.0669x over previous
import functools

import jax
import jax.numpy as jnp
from jax import lax
from jax.experimental import pallas as pl
from jax.experimental.pallas import tpu as pltpu
from jax.experimental.pallas import tpu_sc as plsc

N_Q = 32
N_KV = 8192
D_HEAD = 128
MAX_OFFSET = 512

S_ROWS = 672
ROWS_PER_TILE = S_ROWS // 16
HEAD = 576
CONST_START = 608
CONST_LEN = 64
TAIL = N_KV - HEAD
N_FULL = TAIL // CONST_LEN
REM = TAIL % CONST_LEN
FIRE = 13


def _make_rpe():
    mesh = plsc.VectorSubcoreMesh(core_axis_name="c", subcore_axis_name="s")

    @functools.partial(
        pl.kernel,
        mesh=mesh,
        out_type=jax.ShapeDtypeStruct((N_Q, N_KV, D_HEAD), jnp.float32),
        scratch_types=[
            pltpu.VMEM_SHARED((S_ROWS, D_HEAD), jnp.float32),
            pltpu.SemaphoreType.DMA,
            pltpu.SemaphoreType.DMA,
        ],
    )
    def rpe(table_hbm, out_hbm, s_ref, bsem, wsem):
        c = lax.axis_index("c")
        s = lax.axis_index("s")
        q = c * 16 + s

        u0 = s * ROWS_PER_TILE
        for base in range(0, ROWS_PER_TILE, FIRE):
            burst = [
                pltpu.async_copy(
                    table_hbm.at[jnp.maximum(543 - (u0 + base + r), 0)],
                    s_ref.at[u0 + base + r],
                    bsem,
                )
                for r in range(min(FIRE, ROWS_PER_TILE - base))
            ]
            for cp in burst:
                cp.wait()

        plsc.subcore_barrier()

        writes = [
            pltpu.async_copy(
                s_ref.at[pl.ds(31 - q, HEAD)],
                out_hbm.at[q, pl.ds(0, HEAD)],
                wsem,
            )
        ]
        const_src = s_ref.at[pl.ds(CONST_START, CONST_LEN)]
        for i in range(N_FULL):
            writes.append(
                pltpu.async_copy(
                    const_src,
                    out_hbm.at[q, pl.ds(HEAD + i * CONST_LEN, CONST_LEN)],
                    wsem,
                )
            )
        if REM:
            writes.append(
                pltpu.async_copy(
                    s_ref.at[pl.ds(CONST_START, REM)],
                    out_hbm.at[q, pl.ds(N_KV - REM, REM)],
                    wsem,
                )
            )
        for cp in writes:
            cp.wait()

    return rpe


_rpe = _make_rpe()


def kernel(n_q, n_kv, pos_emb):
    del n_q, n_kv
    return _rpe(pos_emb)


# speedup vs baseline: 2.6835x; 1.0114x over previous
import functools

import jax
import jax.numpy as jnp
from jax import lax
from jax.experimental import pallas as pl
from jax.experimental.pallas import tpu as pltpu
from jax.experimental.pallas import tpu_sc as plsc

N_Q = 32
N_KV = 8192
D_HEAD = 128
MAX_OFFSET = 512

S_ROWS = 640
ROWS_PER_TILE = S_ROWS // 16
HEAD = 576
CONST_START = 608
CONST_LEN = 32
TAIL = N_KV - HEAD
N_FULL = TAIL // CONST_LEN
REM = TAIL % CONST_LEN
FIRE = 13


def _make_rpe():
    mesh = plsc.VectorSubcoreMesh(core_axis_name="c", subcore_axis_name="s")

    @functools.partial(
        pl.kernel,
        mesh=mesh,
        out_type=jax.ShapeDtypeStruct((N_Q, N_KV, D_HEAD), jnp.float32),
        scratch_types=[
            pltpu.VMEM_SHARED((S_ROWS, D_HEAD), jnp.float32),
            pltpu.SemaphoreType.DMA,
            pltpu.SemaphoreType.DMA,
        ],
    )
    def rpe(table_hbm, out_hbm, s_ref, bsem, wsem):
        c = lax.axis_index("c")
        s = lax.axis_index("s")
        q = c * 16 + s

        u0 = s * ROWS_PER_TILE
        for base in range(0, ROWS_PER_TILE, FIRE):
            burst = [
                pltpu.async_copy(
                    table_hbm.at[jnp.maximum(543 - (u0 + base + r), 0)],
                    s_ref.at[u0 + base + r],
                    bsem,
                )
                for r in range(min(FIRE, ROWS_PER_TILE - base))
            ]
            for cp in burst:
                cp.wait()

        plsc.subcore_barrier()

        writes = [
            pltpu.async_copy(
                s_ref.at[pl.ds(31 - q, HEAD)],
                out_hbm.at[q, pl.ds(0, HEAD)],
                wsem,
            )
        ]
        const_src = s_ref.at[pl.ds(CONST_START, CONST_LEN)]
        for i in range(N_FULL):
            writes.append(
                pltpu.async_copy(
                    const_src,
                    out_hbm.at[q, pl.ds(HEAD + i * CONST_LEN, CONST_LEN)],
                    wsem,
                )
            )
        if REM:
            writes.append(
                pltpu.async_copy(
                    s_ref.at[pl.ds(CONST_START, REM)],
                    out_hbm.at[q, pl.ds(N_KV - REM, REM)],
                    wsem,
                )
            )
        for cp in writes:
            cp.wait()

    return rpe


_rpe = _make_rpe()


def kernel(n_q, n_kv, pos_emb):
    del n_q, n_kv
    return _rpe(pos_emb)
